# Initial kernel scaffold; baseline (speedup 1.0000x reference)
import jax
import jax.numpy as jnp
from jax.experimental import pallas as pl


def kernel(lstm_out_list, first_notes, trees, train, embedding, taW1, tab1, taW2, tab2, nW1, nb1, nW2, nb2, tfW1, tfb1, tfW2, tfb2, fW1, fb1, fW2, fb2, tailW, tailb):
    raise NotImplementedError("write your pallas kernel here")


# speedup vs baseline: 20.5739x; 20.5739x over previous
import functools
import math

import jax
import jax.numpy as jnp
import numpy as np
from jax import lax
from jax.experimental import pallas as pl
from jax.experimental.pallas import tpu as pltpu
from jax.experimental.pallas import tpu_sc as plsc

B = 64
T = 200
LSTM_DIM = 256
POS_DIM = 8
EMBED_DIM = 16
N_EMB = 14
NPAIR = 2 * B

_NC, _NS = 2, 16
_NW = _NC * _NS

EMB_ROWS = NPAIR * N_EMB
EMB_PER_W = EMB_ROWS // _NW
PTR_WORKERS = 16
PTR_PER_W = NPAIR // PTR_WORKERS


def _make_pe():
    pe = np.zeros((200, POS_DIM), dtype=np.float32)
    pos = np.arange(200, dtype=np.float32)[:, None]
    div = np.exp(np.arange(0, POS_DIM, 2).astype(np.float32)
                 * (-math.log(10000.0) / POS_DIM))
    pe[:, 0::2] = np.sin(pos * div)
    pe[:, 1::2] = np.cos(pos * div)
    return pe


_PE = _make_pe()

@functools.cache
def _make_sc_gather():
    mesh = plsc.VectorSubcoreMesh(core_axis_name="c", subcore_axis_name="s")

    @functools.partial(
        pl.kernel,
        out_type=(
            jax.ShapeDtypeStruct((NPAIR, LSTM_DIM), jnp.float32),
            jax.ShapeDtypeStruct((EMB_ROWS, EMBED_DIM), jnp.float32),
        ),
        mesh=mesh,
        scratch_types=[
            pltpu.VMEM((PTR_PER_W,), jnp.int32),
            pltpu.VMEM((PTR_PER_W, LSTM_DIM), jnp.float32),
            pltpu.VMEM((EMB_PER_W,), jnp.int32),
            pltpu.VMEM((EMB_PER_W, EMBED_DIM), jnp.float32),
            pltpu.SemaphoreType.DMA,
        ],
        compiler_params=pltpu.CompilerParams(use_tc_tiling_on_sc=False),
    )
    def _sc_gather(lstm_hbm, emb_hbm, iptr_hbm, iemb_hbm,
                   ptr_out, embr_out,
                   iptr_v, ptr_v, iemb_v, embr_v, sem):
        wid = lax.axis_index("s") * _NC + lax.axis_index("c")

        eb = wid * EMB_PER_W
        pltpu.sync_copy(iemb_hbm.at[pl.ds(eb, EMB_PER_W)], iemb_v)
        pltpu.async_copy(emb_hbm.at[iemb_v], embr_v, sem).wait()
        pltpu.sync_copy(embr_v, embr_out.at[pl.ds(eb, EMB_PER_W)])

        @pl.when(wid < PTR_WORKERS)
        def _():
            lb = wid * PTR_PER_W
            pltpu.sync_copy(iptr_hbm.at[pl.ds(lb, PTR_PER_W)], iptr_v)
            pltpu.async_copy(lstm_hbm.at[iptr_v], ptr_v, sem).wait()
            pltpu.sync_copy(ptr_v, ptr_out.at[pl.ds(lb, PTR_PER_W)])

    return _sc_gather


def _mlp_body(ptr_ref, emb_ref, ipe0_ref, ipe1_ref, pe_ref,
              nW1_ref, nb1_ref, nW2_ref, nb2_ref,
              tfW1_ref, tfb1_ref, tfW2_ref, tfb2_ref,
              fW1_ref, fb1_ref, fW2_ref, fb2_ref,
              tailW_ref, tailb_ref, out_ref):
    f32 = jnp.float32
    hi = lax.Precision.HIGHEST

    def mm(a, b):
        return jax.lax.dot_general(a, b, (((1,), (0,)), ((), ())),
                                   precision=hi, preferred_element_type=f32)

    pe = pe_ref[...]
    col = lax.broadcasted_iota(jnp.int32, (NPAIR, 200), 1)
    oh0 = (ipe0_ref[...].reshape(NPAIR, 1) == col).astype(f32)
    oh1 = (ipe1_ref[...].reshape(NPAIR, 1) == col).astype(f32)
    pos0 = mm(oh0, pe)
    pos1 = mm(oh1, pe)

    nW1 = nW1_ref[...]
    h1 = (mm(pos0, nW1[0:POS_DIM, :])
          + mm(pos1, nW1[POS_DIM:2 * POS_DIM, :])
          + mm(emb_ref[...], nW1[2 * POS_DIM:2 * POS_DIM + N_EMB * EMBED_DIM, :])
          + mm(ptr_ref[...], nW1[2 * POS_DIM + N_EMB * EMBED_DIM:, :])
          + nb1_ref[...].reshape(1, -1))
    h1 = jnp.maximum(h1, 0.0)
    h2 = jnp.maximum(mm(h1, nW2_ref[...]) + nb2_ref[...].reshape(1, -1), 0.0)

    a = h2[0:B, :]
    bm = h2[B:NPAIR, :]
    tfW1 = tfW1_ref[...]
    wa = tfW1[0:32, :]
    wb = (tfW1[32:64, :] + tfW1[64:96, :] + tfW1[96:128, :]
          + tfW1[128:160, :] + tfW1[160:192, :])
    s = jnp.maximum(mm(a, wa) + mm(bm, wb) + tfb1_ref[...].reshape(1, -1), 0.0)
    s = jnp.maximum(mm(s, tfW2_ref[...]) + tfb2_ref[...].reshape(1, -1), 0.0)

    x = jnp.maximum(mm(s, fW1_ref[...]) + fb1_ref[...].reshape(1, -1), 0.0)
    x = jnp.maximum(mm(x, fW2_ref[...]) + fb2_ref[...].reshape(1, -1), 0.0)
    out_ref[...] = mm(x, tailW_ref[...]) + tailb_ref[...].reshape(1, -1)


def kernel(lstm_out_list, first_notes, trees, train, embedding,
           taW1, tab1, taW2, tab2, nW1, nb1, nW2, nb2,
           tfW1, tfb1, tfW2, tfb2, fW1, fb1, fW2, fb2, tailW, tailb):
    del first_notes, train, taW1, tab1, taW2, tab2

    trees32 = trees.astype(jnp.int32)
    pairs = jnp.concatenate([trees32[:, T - 1, :], trees32[:, 0, :]], axis=0)
    tree_ids = jnp.tile(jnp.arange(B, dtype=jnp.int32), (2,))
    idx_ptr = tree_ids * T + pairs[:, 16]
    idx_emb = pairs[:, 2:16].reshape(-1)
    ipe0 = pairs[:, 0]
    ipe1 = pairs[:, 1]
    lstm_flat = lstm_out_list.reshape(B * T, LSTM_DIM)

    ptr_rows, emb_rows = _make_sc_gather()(lstm_flat, embedding, idx_ptr, idx_emb)
    emb224 = emb_rows.reshape(NPAIR, N_EMB * EMBED_DIM)

    out = pl.pallas_call(
        _mlp_body,
        out_shape=jax.ShapeDtypeStruct((B, 1), jnp.float32),
    )(ptr_rows, emb224, ipe0, ipe1, _PE,
      nW1, nb1, nW2, nb2, tfW1, tfb1, tfW2, tfb2,
      fW1, fb1, fW2, fb2, tailW, tailb)
    return out
